# baseline (speedup 1.0000x reference)
import functools

import jax
import jax.numpy as jnp
from jax import lax
from jax.experimental import pallas as pl
from jax.experimental.pallas import tpu as pltpu
from jax.experimental.pallas import tpu_sc as plsc

N_NODES = 10000
N_EDGES = 320000
D = 128

NC = 2
NS = 16
NW = NC * NS

N_PAD = NS * 640
CHUNK = 128
CHUNKS_PER_W = 80
STAGES = 2
CHUNKS_PER_STAGE = CHUNKS_PER_W // STAGES
E_PAD = NW * CHUNKS_PER_W * CHUNK
STRIPE = N_PAD // NS
DRAIN_STEPS = STRIPE // CHUNK


def _mm1_body(ih_ref, h_ref, lh_ref, pos_ref, w1a, w1b, w1c, w1d, z_ref):
    acc = jnp.dot(ih_ref[...], w1a[...], preferred_element_type=jnp.float32)
    acc += jnp.dot(h_ref[...], w1b[...], preferred_element_type=jnp.float32)
    acc += jnp.dot(lh_ref[...], w1c[...], preferred_element_type=jnp.float32)
    acc += pos_ref[...] * w1d[...]
    z_ref[...] = acc


def _mm2_body(z_ref, a0_ref, a1_ref, b1_ref, w2_ref, b2_ref, out_ref):
    pre = z_ref[...] + a0_ref[...] + a1_ref[...] + b1_ref[...]
    pre = jnp.maximum(pre, 0.0)
    out_ref[...] = (
        jnp.dot(pre, w2_ref[...], preferred_element_type=jnp.float32) + b2_ref[...]
    )


def _sc_segment_sum(z, src_p, dst_p, zeros_stripe):
    mesh = plsc.VectorSubcoreMesh(core_axis_name="c", subcore_axis_name="s")

    @functools.partial(
        pl.kernel,
        mesh=mesh,
        out_type=jax.ShapeDtypeStruct((NC * N_PAD, D), jnp.float32),
        scratch_types=[
            pltpu.VMEM_SHARED((N_PAD, D), jnp.float32),
            pltpu.VMEM((CHUNKS_PER_STAGE, CHUNK), jnp.int32),
            pltpu.VMEM((CHUNKS_PER_STAGE, CHUNK), jnp.int32),
            pltpu.VMEM((CHUNK, D), jnp.float32),
            pltpu.VMEM((CHUNK, D), jnp.float32),
            pltpu.SemaphoreType.DMA,
            pltpu.SemaphoreType.DMA,
        ],
    )
    def seg_sum(z_hbm, src_hbm, dst_hbm, zeros_hbm, out_hbm,
                acc, src_v, dst_v, rows0, rows1, sem0, sem1):
        cid = lax.axis_index("c")
        sid = lax.axis_index("s")
        wid = sid * NC + cid

        with jax.named_scope("acc_init"):
            pltpu.sync_copy(zeros_hbm, acc.at[pl.ds(sid * STRIPE, STRIPE)])
            plsc.subcore_barrier()

        bufs = (rows0, rows1)
        sems = (sem0, sem1)

        def gather(j, b):
            pltpu.async_copy(z_hbm.at[src_v.at[j]], bufs[b], sems[b])

        def gather_wait(b):
            pltpu.make_async_copy(z_hbm.at[pl.ds(0, CHUNK)], bufs[b], sems[b]).wait()

        for h in range(STAGES):
            with jax.named_scope(f"edges{h}"):
                base = wid * CHUNKS_PER_W + h * CHUNKS_PER_STAGE
                pltpu.sync_copy(src_hbm.at[pl.ds(base, CHUNKS_PER_STAGE)], src_v)
                pltpu.sync_copy(dst_hbm.at[pl.ds(base, CHUNKS_PER_STAGE)], dst_v)
                gather(0, 0)
                gather(1, 1)

                def body(i, carry):
                    for b in range(2):
                        j = 2 * i + b
                        gather_wait(b)
                        pltpu.sync_copy(bufs[b], acc.at[dst_v.at[j]], add=True)

                        @pl.when(j + 2 < CHUNKS_PER_STAGE)
                        def _():
                            gather(j + 2, b)

                    return carry

                lax.fori_loop(0, CHUNKS_PER_STAGE // 2, body, 0)
        with jax.named_scope("drain"):
            plsc.subcore_barrier()

            def drain(c2, carry):
                off = sid * STRIPE + c2 * CHUNK
                pltpu.sync_copy(acc.at[pl.ds(off, CHUNK)], rows0)
                pltpu.sync_copy(rows0, out_hbm.at[pl.ds(cid * N_PAD + off, CHUNK)])
                return carry

            lax.fori_loop(0, DRAIN_STEPS, drain, 0)

    return seg_sum(z, src_p, dst_p, zeros_stripe)


def kernel(input_hidden, hidden, last_hidden, edge_index, pos, W1, b1, W2, b2):
    w1a = W1[0:D]
    w1b = W1[D : 2 * D]
    w1c = W1[2 * D : 3 * D]
    w1d = W1[3 * D : 3 * D + 1]
    pos2d = pos[:, None]
    b1r = b1[None, :]
    b2r = b2[None, :]

    src = edge_index[0]
    dst = edge_index[1]
    pad = E_PAD - N_EDGES
    src_p = jnp.concatenate([src, jnp.zeros((pad,), jnp.int32)]).reshape(
        NW * CHUNKS_PER_W, CHUNK
    )
    dst_p = jnp.concatenate(
        [dst, jnp.full((pad,), N_NODES, jnp.int32)]
    ).reshape(NW * CHUNKS_PER_W, CHUNK)
    zeros_stripe = jnp.zeros((STRIPE, D), jnp.float32)

    blk = 1000
    grid = (N_NODES // blk,)
    row_spec = pl.BlockSpec((blk, D), lambda i: (i, 0))
    w_spec = pl.BlockSpec((D, D), lambda i: (0, 0))
    z = pl.pallas_call(
        _mm1_body,
        grid=grid,
        in_specs=[
            row_spec,
            row_spec,
            row_spec,
            pl.BlockSpec((blk, 1), lambda i: (i, 0)),
            w_spec,
            w_spec,
            w_spec,
            pl.BlockSpec((1, D), lambda i: (0, 0)),
        ],
        out_specs=row_spec,
        out_shape=jax.ShapeDtypeStruct((N_NODES, D), jnp.float32),
    )(input_hidden, hidden, last_hidden, pos2d, w1a, w1b, w1c, w1d)

    agg2 = _sc_segment_sum(z, src_p, dst_p, zeros_stripe)
    agg0 = agg2[0:N_NODES]
    agg1 = agg2[N_PAD : N_PAD + N_NODES]

    out = pl.pallas_call(
        _mm2_body,
        grid=grid,
        in_specs=[
            row_spec,
            row_spec,
            row_spec,
            pl.BlockSpec((1, D), lambda i: (0, 0)),
            w_spec,
            pl.BlockSpec((1, D), lambda i: (0, 0)),
        ],
        out_specs=row_spec,
        out_shape=jax.ShapeDtypeStruct((N_NODES, D), jnp.float32),
    )(z, agg0, agg1, b1r, W2, b2r)
    return out


# speedup vs baseline: 2.6958x; 2.6958x over previous
import functools

import jax
import jax.numpy as jnp
from jax import lax
from jax.experimental import pallas as pl
from jax.experimental.pallas import tpu as pltpu
from jax.experimental.pallas import tpu_sc as plsc

N_NODES = 10000
N_EDGES = 320000
D = 128

NC = 2
NS = 16
NW = NC * NS

N_PAD = NS * 640
CHUNK = 128
CHUNKS_PER_W = 80
STAGES = 2
CHUNKS_PER_STAGE = CHUNKS_PER_W // STAGES
E_PAD = NW * CHUNKS_PER_W * CHUNK
STRIPE = N_PAD // NS
DRAIN_STEPS = STRIPE // CHUNK


def _mm1_body(ih_ref, h_ref, lh_ref, pos_ref, w1a, w1b, w1c, w1d, z_ref):
    acc = jnp.dot(ih_ref[...], w1a[...], preferred_element_type=jnp.float32)
    acc += jnp.dot(h_ref[...], w1b[...], preferred_element_type=jnp.float32)
    acc += jnp.dot(lh_ref[...], w1c[...], preferred_element_type=jnp.float32)
    acc += pos_ref[...] * w1d[...]
    z_ref[...] = acc


def _mm2_body(z_ref, a0_ref, a1_ref, b1_ref, w2_ref, b2_ref, out_ref):
    pre = z_ref[...] + a0_ref[...] + a1_ref[...] + b1_ref[...]
    pre = jnp.maximum(pre, 0.0)
    out_ref[...] = (
        jnp.dot(pre, w2_ref[...], preferred_element_type=jnp.float32) + b2_ref[...]
    )


def _sc_segment_sum(z, src_p, dst_p, zeros_stripe):
    mesh = plsc.VectorSubcoreMesh(core_axis_name="c", subcore_axis_name="s")

    @functools.partial(
        pl.kernel,
        mesh=mesh,
        out_type=jax.ShapeDtypeStruct((NC * N_PAD, D), jnp.float32),
        scratch_types=[
            pltpu.VMEM_SHARED((N_PAD, D), jnp.float32),
            pltpu.VMEM((CHUNKS_PER_STAGE, CHUNK), jnp.int32),
            pltpu.VMEM((CHUNKS_PER_STAGE, CHUNK), jnp.int32),
            pltpu.VMEM((CHUNK, D), jnp.float32),
            pltpu.VMEM((CHUNK, D), jnp.float32),
            pltpu.SemaphoreType.DMA,
            pltpu.SemaphoreType.DMA,
        ],
    )
    def seg_sum(z_hbm, src_hbm, dst_hbm, zeros_hbm, out_hbm,
                acc, src_v, dst_v, rows0, rows1, sem0, sem1):
        cid = lax.axis_index("c")
        sid = lax.axis_index("s")
        wid = sid * NC + cid

        with jax.named_scope("acc_init"):
            pltpu.sync_copy(zeros_hbm, acc.at[pl.ds(sid * STRIPE, STRIPE)])
            plsc.subcore_barrier()

        bufs = (rows0, rows1)
        sems = (sem0, sem1)

        def gather(j, b):
            pltpu.async_copy(z_hbm.at[src_v.at[j]], bufs[b], sems[b])

        def gather_wait(b):
            pltpu.make_async_copy(z_hbm.at[pl.ds(0, CHUNK)], bufs[b], sems[b]).wait()

        for h in range(STAGES):
            with jax.named_scope(f"edges{h}"):
                base = wid * CHUNKS_PER_W + h * CHUNKS_PER_STAGE
                pltpu.sync_copy(src_hbm.at[pl.ds(base, CHUNKS_PER_STAGE)], src_v)
                pltpu.sync_copy(dst_hbm.at[pl.ds(base, CHUNKS_PER_STAGE)], dst_v)
                gather(0, 0)
                gather(1, 1)

                def body(i, carry):
                    for b in range(2):
                        j = 2 * i + b
                        gather_wait(b)
                        pltpu.sync_copy(bufs[b], acc.at[dst_v.at[j]], add=True)

                        @pl.when(j + 2 < CHUNKS_PER_STAGE)
                        def _():
                            gather(j + 2, b)

                    return carry

                lax.fori_loop(0, CHUNKS_PER_STAGE // 2, body, 0)
        with jax.named_scope("drain"):
            plsc.subcore_barrier()

            def drain(c2, carry):
                off = sid * STRIPE + c2 * CHUNK
                pltpu.sync_copy(acc.at[pl.ds(off, CHUNK)], rows0)
                pltpu.sync_copy(rows0, out_hbm.at[pl.ds(cid * N_PAD + off, CHUNK)])
                return carry

            lax.fori_loop(0, DRAIN_STEPS, drain, 0)

    return seg_sum(z, src_p, dst_p, zeros_stripe)


def kernel(input_hidden, hidden, last_hidden, edge_index, pos, W1, b1, W2, b2):
    w1a = W1[0:D]
    w1b = W1[D : 2 * D]
    w1c = W1[2 * D : 3 * D]
    w1d = W1[3 * D : 3 * D + 1]
    pos2d = pos[:, None]
    b1r = b1[None, :]
    b2r = b2[None, :]

    src = edge_index[0]
    dst = edge_index[1]
    pad = E_PAD - N_EDGES
    pad_ids = jnp.arange(pad, dtype=jnp.int32)
    src_p = jnp.concatenate([src, pad_ids % N_NODES]).reshape(
        NW * CHUNKS_PER_W, CHUNK
    )
    dst_p = jnp.concatenate(
        [dst, N_NODES + pad_ids % (N_PAD - N_NODES)]
    ).reshape(NW * CHUNKS_PER_W, CHUNK)
    zeros_stripe = jnp.zeros((STRIPE, D), jnp.float32)

    blk = 1000
    grid = (N_NODES // blk,)
    row_spec = pl.BlockSpec((blk, D), lambda i: (i, 0))
    w_spec = pl.BlockSpec((D, D), lambda i: (0, 0))
    z = pl.pallas_call(
        _mm1_body,
        grid=grid,
        in_specs=[
            row_spec,
            row_spec,
            row_spec,
            pl.BlockSpec((blk, 1), lambda i: (i, 0)),
            w_spec,
            w_spec,
            w_spec,
            pl.BlockSpec((1, D), lambda i: (0, 0)),
        ],
        out_specs=row_spec,
        out_shape=jax.ShapeDtypeStruct((N_NODES, D), jnp.float32),
    )(input_hidden, hidden, last_hidden, pos2d, w1a, w1b, w1c, w1d)

    agg2 = _sc_segment_sum(z, src_p, dst_p, zeros_stripe)
    agg0 = agg2[0:N_NODES]
    agg1 = agg2[N_PAD : N_PAD + N_NODES]

    out = pl.pallas_call(
        _mm2_body,
        grid=grid,
        in_specs=[
            row_spec,
            row_spec,
            row_spec,
            pl.BlockSpec((1, D), lambda i: (0, 0)),
            w_spec,
            pl.BlockSpec((1, D), lambda i: (0, 0)),
        ],
        out_specs=row_spec,
        out_shape=jax.ShapeDtypeStruct((N_NODES, D), jnp.float32),
    )(z, agg0, agg1, b1r, W2, b2r)
    return out


# speedup vs baseline: 2.8159x; 1.0445x over previous
import functools

import jax
import jax.numpy as jnp
import numpy as np
from jax import lax
from jax.experimental import pallas as pl
from jax.experimental.pallas import tpu as pltpu
from jax.experimental.pallas import tpu_sc as plsc

N_NODES = 10000
N_EDGES = 320000
D = 128

NC = 2
NS = 16
NW = NC * NS

N_PAD = NS * 640
CHUNK = 128
CHUNKS_PER_W = 80
STAGES = 2
CHUNKS_PER_STAGE = CHUNKS_PER_W // STAGES
E_PAD = NW * CHUNKS_PER_W * CHUNK
STRIPE = N_PAD // NS
DRAIN_STEPS = STRIPE // CHUNK

_PAD_E = E_PAD - N_EDGES
_SRC_TAIL = np.arange(_PAD_E, dtype=np.int32) % N_NODES
_DST_TAIL = N_NODES + np.arange(_PAD_E, dtype=np.int32) % (N_PAD - N_NODES)
_ZEROS_STRIPE = np.zeros((STRIPE, D), np.float32)


def _mm1_body(ih_ref, h_ref, lh_ref, pos_ref, w1a, w1b, w1c, w1d, z_ref):
    acc = jnp.dot(ih_ref[...], w1a[...], preferred_element_type=jnp.float32)
    acc += jnp.dot(h_ref[...], w1b[...], preferred_element_type=jnp.float32)
    acc += jnp.dot(lh_ref[...], w1c[...], preferred_element_type=jnp.float32)
    acc += pos_ref[...] * w1d[...]
    z_ref[...] = acc


def _mm2_body(z_ref, a0_ref, a1_ref, b1_ref, w2_ref, b2_ref, out_ref):
    pre = z_ref[...] + a0_ref[0] + a1_ref[0] + b1_ref[...]
    pre = jnp.maximum(pre, 0.0)
    out_ref[...] = (
        jnp.dot(pre, w2_ref[...], preferred_element_type=jnp.float32) + b2_ref[...]
    )


def _sc_segment_sum(z, src_p, dst_p, zeros_stripe):
    mesh = plsc.VectorSubcoreMesh(core_axis_name="c", subcore_axis_name="s")

    @functools.partial(
        pl.kernel,
        mesh=mesh,
        out_type=jax.ShapeDtypeStruct((NC, N_PAD, D), jnp.float32),
        scratch_types=[
            pltpu.VMEM_SHARED((N_PAD, D), jnp.float32),
            pltpu.VMEM((CHUNKS_PER_STAGE, CHUNK), jnp.int32),
            pltpu.VMEM((CHUNKS_PER_STAGE, CHUNK), jnp.int32),
            pltpu.VMEM((CHUNK, D), jnp.float32),
            pltpu.VMEM((CHUNK, D), jnp.float32),
            pltpu.SemaphoreType.DMA,
            pltpu.SemaphoreType.DMA,
        ],
    )
    def seg_sum(z_hbm, src_hbm, dst_hbm, zeros_hbm, out_hbm,
                acc, src_v, dst_v, rows0, rows1, sem0, sem1):
        cid = lax.axis_index("c")
        sid = lax.axis_index("s")
        wid = sid * NC + cid

        with jax.named_scope("acc_init"):
            init_cp = pltpu.async_copy(
                zeros_hbm, acc.at[pl.ds(sid * STRIPE, STRIPE)], sem0
            )
            base0 = wid * CHUNKS_PER_W
            pltpu.sync_copy(src_hbm.at[pl.ds(base0, CHUNKS_PER_STAGE)], src_v)
            pltpu.sync_copy(dst_hbm.at[pl.ds(base0, CHUNKS_PER_STAGE)], dst_v)
            init_cp.wait()
            plsc.subcore_barrier()

        bufs = (rows0, rows1)
        sems = (sem0, sem1)

        def gather(j, b):
            pltpu.async_copy(z_hbm.at[src_v.at[j]], bufs[b], sems[b])

        def gather_wait(b):
            pltpu.make_async_copy(z_hbm.at[pl.ds(0, CHUNK)], bufs[b], sems[b]).wait()

        for h in range(STAGES):
            with jax.named_scope(f"edges{h}"):
                if h > 0:
                    base = wid * CHUNKS_PER_W + h * CHUNKS_PER_STAGE
                    pltpu.sync_copy(
                        src_hbm.at[pl.ds(base, CHUNKS_PER_STAGE)], src_v
                    )
                    pltpu.sync_copy(
                        dst_hbm.at[pl.ds(base, CHUNKS_PER_STAGE)], dst_v
                    )
                gather(0, 0)
                gather(1, 1)

                def body(i, carry):
                    for b in range(2):
                        j = 2 * i + b
                        gather_wait(b)
                        pltpu.sync_copy(bufs[b], acc.at[dst_v.at[j]], add=True)

                        @pl.when(j + 2 < CHUNKS_PER_STAGE)
                        def _():
                            gather(j + 2, b)

                    return carry

                lax.fori_loop(0, CHUNKS_PER_STAGE // 2, body, 0)
        with jax.named_scope("drain"):
            plsc.subcore_barrier()
            pltpu.sync_copy(
                acc.at[pl.ds(sid * STRIPE, STRIPE)],
                out_hbm.at[cid, pl.ds(sid * STRIPE, STRIPE)],
            )

    return seg_sum(z, src_p, dst_p, zeros_stripe)


def kernel(input_hidden, hidden, last_hidden, edge_index, pos, W1, b1, W2, b2):
    w1a = W1[0:D]
    w1b = W1[D : 2 * D]
    w1c = W1[2 * D : 3 * D]
    w1d = W1[3 * D : 3 * D + 1]
    pos2d = pos[:, None]
    b1r = b1[None, :]
    b2r = b2[None, :]

    src_p = jnp.concatenate([edge_index[0], jnp.asarray(_SRC_TAIL)]).reshape(
        NW * CHUNKS_PER_W, CHUNK
    )
    dst_p = jnp.concatenate([edge_index[1], jnp.asarray(_DST_TAIL)]).reshape(
        NW * CHUNKS_PER_W, CHUNK
    )

    blk = 1000
    grid = (N_NODES // blk,)
    row_spec = pl.BlockSpec((blk, D), lambda i: (i, 0))
    w_spec = pl.BlockSpec((D, D), lambda i: (0, 0))
    bias_spec = pl.BlockSpec((1, D), lambda i: (0, 0))
    z = pl.pallas_call(
        _mm1_body,
        grid=grid,
        in_specs=[
            row_spec,
            row_spec,
            row_spec,
            pl.BlockSpec((blk, 1), lambda i: (i, 0)),
            w_spec,
            w_spec,
            w_spec,
            bias_spec,
        ],
        out_specs=row_spec,
        out_shape=jax.ShapeDtypeStruct((N_NODES, D), jnp.float32),
    )(input_hidden, hidden, last_hidden, pos2d, w1a, w1b, w1c, w1d)

    agg2 = _sc_segment_sum(z, src_p, dst_p, jnp.asarray(_ZEROS_STRIPE))

    out = pl.pallas_call(
        _mm2_body,
        grid=grid,
        in_specs=[
            row_spec,
            pl.BlockSpec((1, blk, D), lambda i: (0, i, 0)),
            pl.BlockSpec((1, blk, D), lambda i: (1, i, 0)),
            bias_spec,
            w_spec,
            bias_spec,
        ],
        out_specs=row_spec,
        out_shape=jax.ShapeDtypeStruct((N_NODES, D), jnp.float32),
    )(z, agg2, agg2, b1r, W2, b2r)
    return out
